# Initial kernel scaffold; baseline (speedup 1.0000x reference)
import jax
import jax.numpy as jnp
from jax.experimental import pallas as pl


def kernel(x, edge_index, Wr, W1, b1, W2, b2):
    raise NotImplementedError("write your pallas kernel here")


# speedup vs baseline: 4.1703x; 4.1703x over previous
import functools

import jax
import jax.numpy as jnp
from jax import lax
from jax.experimental import pallas as pl
from jax.experimental.pallas import tpu as pltpu
from jax.experimental.pallas import tpu_sc as plsc

N = 10000
E = 320000
D = 128
NEXP = 8
NLAYERS = 2
TAU = 0.8

NC = 2
NS = 16
NW = NC * NS
CHUNK = 128
CPT = -(-E // (NW * CHUNK))
E_PAD = NW * CPT * CHUNK
N_PAD = 10240
RPS = N_PAD // NS


def _mesh():
  return plsc.VectorSubcoreMesh(
      core_axis_name="c", subcore_axis_name="s", num_cores=NC, num_subcores=NS)


def _zero_block(ref):
  @pl.loop(0, CHUNK)
  def _z(j):
    for k in range(D // 16):
      ref[j, pl.ds(k * 16, 16)] = jnp.zeros((16,), jnp.float32)


@functools.lru_cache(maxsize=None)
def _make_agg_call():
  def body(h_hbm, srcm, dstm, agg_out, acc_sh, sidx, didx, rows, sem):
    c = lax.axis_index("c")
    s = lax.axis_index("s")
    wid = c * NS + s

    _zero_block(rows)
    r0 = s * RPS
    for i in range(RPS // CHUNK):
      pltpu.sync_copy(rows, acc_sh.at[pl.ds(r0 + i * CHUNK, CHUNK)])
    plsc.subcore_barrier()

    @pl.loop(0, CPT)
    def _edges(j):
      pltpu.sync_copy(srcm.at[wid, j], sidx)
      pltpu.sync_copy(dstm.at[wid, j], didx)
      pltpu.async_copy(h_hbm.at[sidx], rows, sem).wait()
      pltpu.sync_copy(rows, acc_sh.at[didx], add=True)

    plsc.subcore_barrier()
    pltpu.sync_copy(acc_sh.at[pl.ds(r0, RPS)], agg_out.at[c, pl.ds(r0, RPS)])

  return pl.kernel(
      body,
      out_type=[jax.ShapeDtypeStruct((NC, N_PAD, D), jnp.float32)],
      mesh=_mesh(),
      scratch_types=[
          pltpu.VMEM_SHARED((N_PAD, D), jnp.float32),
          pltpu.VMEM((CHUNK,), jnp.int32),
          pltpu.VMEM((CHUNK,), jnp.int32),
          pltpu.VMEM((CHUNK, D), jnp.float32),
          pltpu.SemaphoreType.DMA,
      ],
  )


@functools.lru_cache(maxsize=None)
def _make_deg_call():
  def body(dstm, deg_out, acc_sh, didx, rows, sem):
    c = lax.axis_index("c")
    s = lax.axis_index("s")
    wid = c * NS + s

    _zero_block(rows)
    r0 = s * RPS
    for i in range(RPS // CHUNK):
      pltpu.sync_copy(rows, acc_sh.at[pl.ds(r0 + i * CHUNK, CHUNK)])

    @pl.loop(0, CHUNK)
    def _ones(j):
      for k in range(D // 16):
        rows[j, pl.ds(k * 16, 16)] = jnp.full((16,), 1.0, jnp.float32)

    plsc.subcore_barrier()

    @pl.loop(0, CPT)
    def _edges(j):
      pltpu.sync_copy(dstm.at[wid, j], didx)
      pltpu.sync_copy(rows, acc_sh.at[didx], add=True)

    plsc.subcore_barrier()
    pltpu.sync_copy(acc_sh.at[pl.ds(r0, RPS)], deg_out.at[c, pl.ds(r0, RPS)])

  return pl.kernel(
      body,
      out_type=[jax.ShapeDtypeStruct((NC, N_PAD, D), jnp.float32)],
      mesh=_mesh(),
      scratch_types=[
          pltpu.VMEM_SHARED((N_PAD, D), jnp.float32),
          pltpu.VMEM((CHUNK,), jnp.int32),
          pltpu.VMEM((CHUNK, D), jnp.float32),
          pltpu.SemaphoreType.DMA,
      ],
  )


BN = 1000


def _make_tc_call(final):
  def tc_body(h, a, dg, g, wr, w1, b1, w2, b2, out):
    deg = jnp.maximum(dg[0, :, :1] + dg[1, :, :1], 1.0)
    agg = (a[0] + a[1]) / deg
    xin = jnp.concatenate([h[...], agg], axis=1)
    logits = jnp.dot(xin, wr[...], preferred_element_type=jnp.float32)
    z = (logits + g[...]) * (1.0 / TAU)
    z = z - jnp.max(z, axis=1, keepdims=True)
    ez = jnp.exp(z)
    gates = ez / jnp.sum(ez, axis=1, keepdims=True)
    acc = jnp.zeros((BN, D), jnp.float32)
    for e in range(NEXP):
      he = jnp.maximum(
          jnp.dot(xin, w1[e], preferred_element_type=jnp.float32)
          + b1[e][None, :], 0.0)
      oe = jnp.dot(he, w2[e], preferred_element_type=jnp.float32) + b2[e][None, :]
      acc = acc + gates[:, e:e + 1] * oe
    out[...] = acc if final else jnp.maximum(acc, 0.0)

  return pl.pallas_call(
      tc_body,
      grid=(N // BN,),
      in_specs=[
          pl.BlockSpec((BN, D), lambda i: (i, 0)),
          pl.BlockSpec((NC, BN, D), lambda i: (0, i, 0)),
          pl.BlockSpec((NC, BN, D), lambda i: (0, i, 0)),
          pl.BlockSpec((BN, NEXP), lambda i: (i, 0)),
          pl.BlockSpec((2 * D, NEXP), lambda i: (0, 0)),
          pl.BlockSpec((NEXP, 2 * D, D), lambda i: (0, 0, 0)),
          pl.BlockSpec((NEXP, D), lambda i: (0, 0)),
          pl.BlockSpec((NEXP, D, D), lambda i: (0, 0, 0)),
          pl.BlockSpec((NEXP, D), lambda i: (0, 0)),
      ],
      out_specs=pl.BlockSpec((BN, D), lambda i: (i, 0)),
      out_shape=jax.ShapeDtypeStruct((N, D), jnp.float32),
  )


def kernel(x, edge_index, Wr, W1, b1, W2, b2):
  src = edge_index[0]
  dst = edge_index[1]
  pad = E_PAD - E
  srcm = jnp.concatenate([src, jnp.zeros((pad,), jnp.int32)]).reshape(
      NW, CPT, CHUNK)
  dstm = jnp.concatenate([dst, jnp.full((pad,), N, jnp.int32)]).reshape(
      NW, CPT, CHUNK)
  gkey = jax.random.key(42)
  (degp,) = _make_deg_call()(dstm)
  degp = degp[:, :N]
  h = x
  for l in range(NLAYERS):
    (aggp,) = _make_agg_call()(h, srcm, dstm)
    aggp = aggp[:, :N]
    u = jax.random.uniform(jax.random.fold_in(gkey, l), (N, NEXP),
                           minval=1e-6, maxval=1.0 - 1e-6)
    g = -jnp.log(-jnp.log(u))
    tc = _make_tc_call(l == NLAYERS - 1)
    h = tc(h, aggp, degp, g, Wr[l], W1[l], b1[l], W2[l], b2[l])
  return h
